# baseline (speedup 1.0000x reference)
import jax
import jax.numpy as jnp
from jax import lax
from jax.experimental import pallas as pl
from jax.experimental.pallas import tpu as pltpu
from jax.experimental.pallas import tpu_sc as plsc

B = 8
NSIDE_IN = 512
N_IN = 12 * NSIDE_IN * NSIDE_IN
N_OUT = N_IN // 4

NUM_CORES = 2
NUM_SUBCORES = 16
NW = NUM_CORES * NUM_SUBCORES
WPR = NW // B
OUT_PER_W = N_OUT // WPR
IN_PER_W = OUT_PER_W * 4

CHUNK_OUT = 8192
CHUNK_IN = CHUNK_OUT * 4
NBUF = 3
N_CHUNKS = OUT_PER_W // CHUNK_OUT
UNROLL = 8
PAD = 64


def _sc_body(x_hbm, out_hbm, in_v0, in_v1, in_v2,
             out_v0, out_v1, out_v2,
             si0, si1, si2, so0, so1, so2):
    wid = lax.axis_index("s") * NUM_CORES + lax.axis_index("c")
    row = wid // WPR
    in_col = (wid % WPR) * IN_PER_W
    out_col = (wid % WPR) * OUT_PER_W
    iota = lax.iota(jnp.int32, 16)
    idx = [iota * 4 + k for k in range(4)]
    in_bufs = [in_v0, in_v1, in_v2]
    out_bufs = [out_v0, out_v1, out_v2]
    sin = [si0, si1, si2]
    sout = [so0, so1, so2]

    def start_in(g, b):
        pltpu.async_copy(
            x_hbm.at[row, pl.ds(in_col + g * CHUNK_IN, CHUNK_IN)],
            in_bufs[b].at[pl.ds(0, CHUNK_IN)], sin[b])

    for b in range(NBUF):
        start_in(b, b)

    def chunk_group(t, carry):
        for b in range(NBUF):
            g = t * NBUF + b
            pltpu.make_async_copy(
                x_hbm.at[0, pl.ds(0, CHUNK_IN)],
                in_bufs[b].at[pl.ds(0, CHUNK_IN)], sin[b]).wait()

            @pl.when(t > 0)
            def _():
                pltpu.make_async_copy(
                    out_bufs[b], out_hbm.at[0, pl.ds(0, CHUNK_OUT)],
                    sout[b]).wait()

            def load4(o):
                src = in_bufs[b].at[pl.ds(o, 64)]
                return (plsc.load_gather(src, [idx[0]]),
                        plsc.load_gather(src, [idx[1]]),
                        plsc.load_gather(src, [idx[2]]),
                        plsc.load_gather(src, [idx[3]]))

            def vec_body(jj, carry2):
                va, vb, vc, vd = carry2
                base = jj * (64 * UNROLL)
                for u in range(UNROLL):
                    na, nb, nc, nd = load4(base + (u + 1) * 64)
                    out_bufs[b][pl.ds(jj * (16 * UNROLL) + u * 16, 16)] = (
                        jnp.maximum(jnp.maximum(va, vb),
                                    jnp.maximum(vc, vd)))
                    va, vb, vc, vd = na, nb, nc, nd
                return va, vb, vc, vd

            lax.fori_loop(0, CHUNK_OUT // (16 * UNROLL), vec_body, load4(0))

            @pl.when(t < (N_CHUNKS // NBUF - 1))
            def _():
                start_in(g + NBUF, b)

            pltpu.async_copy(
                out_bufs[b],
                out_hbm.at[row, pl.ds(out_col + g * CHUNK_OUT, CHUNK_OUT)],
                sout[b])
        return carry

    lax.fori_loop(0, N_CHUNKS // NBUF, chunk_group, 0)

    for b in range(NBUF):
        pltpu.make_async_copy(
            out_bufs[b], out_hbm.at[0, pl.ds(0, CHUNK_OUT)], sout[b]).wait()


def kernel(x):
    mesh = plsc.VectorSubcoreMesh(core_axis_name="c", subcore_axis_name="s")
    return pl.kernel(
        _sc_body,
        mesh=mesh,
        compiler_params=pltpu.CompilerParams(needs_layout_passes=False,
                                             use_tc_tiling_on_sc=False),
        out_type=jax.ShapeDtypeStruct((B, N_OUT), jnp.float32),
        scratch_types=(
            [pltpu.VMEM((CHUNK_IN + PAD,), jnp.float32)] * NBUF
            + [pltpu.VMEM((CHUNK_OUT,), jnp.float32)] * NBUF
            + [pltpu.SemaphoreType.DMA] * (2 * NBUF)
        ),
    )(x)


# speedup vs baseline: 9.6841x; 9.6841x over previous
import jax
import jax.numpy as jnp
from jax import lax
from jax.experimental import pallas as pl
from jax.experimental.pallas import tpu as pltpu
from jax.experimental.pallas import tpu_sc as plsc

B = 8
NSIDE_IN = 512
N_IN = 12 * NSIDE_IN * NSIDE_IN
N_OUT = N_IN // 4

NUM_CORES = 2
NUM_SUBCORES = 16
NW = NUM_CORES * NUM_SUBCORES
WPR = NW // B
OUT_PER_W = N_OUT // WPR
IN_PER_W = OUT_PER_W * 4

CHUNK_OUT = 8192
CHUNK_IN = CHUNK_OUT * 4
NBUF = 3
N_CHUNKS = OUT_PER_W // CHUNK_OUT
UNROLL = 8
PAD = 64


def _sc_body(x_hbm, out_hbm, in_v0, in_v1, in_v2,
             out_v0, out_v1, out_v2,
             si0, si1, si2, so0, so1, so2):
    wid = lax.axis_index("s") * NUM_CORES + lax.axis_index("c")
    row = wid // WPR
    in_col = (wid % WPR) * IN_PER_W
    out_col = (wid % WPR) * OUT_PER_W
    iota = lax.iota(jnp.int32, 16)
    idx = [iota * 4 + k for k in range(4)]
    in_bufs = [in_v0, in_v1, in_v2]
    out_bufs = [out_v0, out_v1, out_v2]
    sin = [si0, si1, si2]
    sout = [so0, so1, so2]

    def start_in(g, b):
        pltpu.async_copy(
            x_hbm.at[row, pl.ds(in_col + g * CHUNK_IN, CHUNK_IN)],
            in_bufs[b].at[pl.ds(0, CHUNK_IN)], sin[b])

    for b in range(NBUF):
        start_in(b, b)

    def chunk_group(t, carry):
        for b in range(NBUF):
            g = t * NBUF + b
            pltpu.make_async_copy(
                x_hbm.at[0, pl.ds(0, CHUNK_IN)],
                in_bufs[b].at[pl.ds(0, CHUNK_IN)], sin[b]).wait()

            @pl.when(t > 0)
            def _():
                pltpu.make_async_copy(
                    out_bufs[b], out_hbm.at[0, pl.ds(0, CHUNK_OUT)],
                    sout[b]).wait()

            def load4(o):
                src = in_bufs[b].at[pl.ds(o, 64)]
                return (plsc.load_gather(src, [idx[0]]),
                        plsc.load_gather(src, [idx[1]]),
                        plsc.load_gather(src, [idx[2]]),
                        plsc.load_gather(src, [idx[3]]))

            def vec_body(jj, carry2):
                va, vb, vc, vd = carry2
                base = jj * (64 * UNROLL)
                for u in range(UNROLL):
                    na, nb, nc, nd = load4(base + (u + 1) * 64)
                    out_bufs[b][pl.ds(jj * (16 * UNROLL) + u * 16, 16)] = (
                        jnp.maximum(jnp.maximum(va, vb),
                                    jnp.maximum(vc, vd)))
                    va, vb, vc, vd = na, nb, nc, nd
                return va, vb, vc, vd

            lax.fori_loop(0, CHUNK_OUT // (16 * UNROLL), vec_body, load4(0))

            @pl.when(t < (N_CHUNKS // NBUF - 1))
            def _():
                start_in(g + NBUF, b)

            pltpu.async_copy(
                out_bufs[b],
                out_hbm.at[row, pl.ds(out_col + g * CHUNK_OUT, CHUNK_OUT)],
                sout[b])
        return carry

    lax.fori_loop(0, N_CHUNKS // NBUF, chunk_group, 0)

    for b in range(NBUF):
        pltpu.make_async_copy(
            out_bufs[b], out_hbm.at[0, pl.ds(0, CHUNK_OUT)], sout[b]).wait()


def kernel(x):
    mesh = plsc.VectorSubcoreMesh(core_axis_name="c", subcore_axis_name="s")
    return pl.kernel(
        _sc_body,
        mesh=mesh,
        compiler_params=pltpu.CompilerParams(needs_layout_passes=False),
        out_type=jax.ShapeDtypeStruct((B, N_OUT), jnp.float32),
        scratch_types=(
            [pltpu.VMEM((CHUNK_IN + PAD,), jnp.float32)] * NBUF
            + [pltpu.VMEM((CHUNK_OUT,), jnp.float32)] * NBUF
            + [pltpu.SemaphoreType.DMA] * (2 * NBUF)
        ),
    )(x)
